# baseline (speedup 1.0000x reference)
import jax
import jax.numpy as jnp
from jax import lax
from jax.experimental import pallas as pl
from jax.experimental.pallas import tpu as pltpu
from jax.experimental.pallas import tpu_sc as plsc

_N = 10000
_D = 128
_ATT = 64
_E = 320000
_L = 16
_NS = 16
_CPT = 156
_TAIL = _E - _CPT * _NS * 128
_ZR = 624


def _sc_spmm(xa, xb, edges):
    mesh = plsc.VectorSubcoreMesh(core_axis_name="c", subcore_axis_name="s")
    zr = jnp.zeros((_ZR, _D), jnp.float32)
    out_type = tuple(jax.ShapeDtypeStruct((_N, _D), jnp.float32) for _ in range(4))
    scratch = [
        pltpu.VMEM_SHARED((_N, _D), jnp.float32),
        pltpu.VMEM((2, 128, _D), jnp.float32),
        pltpu.VMEM((2, 128), jnp.int32),
        pltpu.VMEM((2, 128), jnp.float32),
        pltpu.VMEM((2, 128), jnp.int32),
        pltpu.SemaphoreType.DMA,
        pltpu.SemaphoreType.DMA,
    ]

    def body(xa_h, xb_h, z_h,
             c0, v0, r0, c1, v1, r1, c2, v2, r2, c3, v3, r3,
             o0, o1, o2, o3,
             acc, gbuf, colv, valv, rowv, sem, semi):
        cid = lax.axis_index("c")
        sid = lax.axis_index("s")

        def do_rel(x_h, cols_h, vals_h, rows_h, out_h):
            pltpu.sync_copy(z_h, acc.at[pl.ds(sid * _ZR, _ZR)])

            @pl.when(sid == 0)
            def _():
                pltpu.sync_copy(z_h.at[pl.ds(0, 16)],
                                acc.at[pl.ds(_NS * _ZR, 16)])

            plsc.subcore_barrier()

            def idx_load(base, slot, copy):
                copy(cols_h.at[pl.ds(base, 128)], colv.at[slot])
                copy(vals_h.at[pl.ds(base, 128)], valv.at[slot])
                copy(rows_h.at[pl.ds(base, 128)], rowv.at[slot])

            def scale_scatter(b):
                def grp(i, c2_):
                    off = pl.multiple_of(i * _L, _L)
                    v16 = valv[b, pl.ds(off, _L)]
                    for e in range(_L):
                        row = off + e
                        sv = v16[e]
                        for f in range(_D // _L):
                            sl = pl.ds(f * _L, _L)
                            gbuf[b, row, sl] = gbuf[b, row, sl] * sv
                    return c2_

                lax.fori_loop(0, 128 // _L, grp, 0)
                pltpu.sync_copy(gbuf.at[b], acc.at[rowv.at[b]], add=True)

            j0 = sid * _CPT * 128
            idx_load(j0, 0, pltpu.sync_copy)
            pltpu.async_copy(x_h.at[colv.at[0]], gbuf.at[0], sem)

            def step(k, carry):
                b = k & 1
                do_pf = k + 1 < _CPT
                nxt = jnp.where(do_pf, j0 + (k + 1) * 128, j0)
                pltpu.make_async_copy(x_h.at[colv.at[b]], gbuf.at[b],
                                      sem).wait()

                @pl.when(do_pf)
                def _():
                    idx_load(pl.multiple_of(nxt, 128), 1 - b,
                             lambda s_, d_: pltpu.async_copy(s_, d_, semi))

                scale_scatter(b)

                @pl.when(do_pf)
                def _():
                    idx_load(0, 1 - b,
                             lambda s_, d_:
                             pltpu.make_async_copy(s_, d_, semi).wait())
                    pltpu.async_copy(x_h.at[colv.at[1 - b]], gbuf.at[1 - b],
                                     sem)

                return carry

            lax.fori_loop(0, _CPT, step, 0)

            @pl.when(sid < _TAIL // 128)
            def _():
                base = _CPT * _NS * 128 + sid * 128
                idx_load(base, 0, pltpu.sync_copy)
                pltpu.async_copy(x_h.at[colv.at[0]], gbuf.at[0], sem).wait()
                scale_scatter(0)

            plsc.subcore_barrier()
            pltpu.sync_copy(acc.at[pl.ds(sid * _ZR, _ZR)],
                            out_h.at[pl.ds(sid * _ZR, _ZR)])

            @pl.when(sid == 0)
            def _():
                pltpu.sync_copy(acc.at[pl.ds(_NS * _ZR, 16)],
                                out_h.at[pl.ds(_NS * _ZR, 16)])

            plsc.subcore_barrier()

        @pl.when(cid == 0)
        def _():
            do_rel(xa_h, c0, v0, r0, o0)
            do_rel(xb_h, c1, v1, r1, o1)

        @pl.when(cid == 1)
        def _():
            do_rel(xa_h, c2, v2, r2, o2)
            do_rel(xb_h, c3, v3, r3, o3)

    f = pl.kernel(body, out_type=out_type, mesh=mesh, scratch_types=scratch)
    (c0, v0, r0), (c1, v1, r1), (c2, v2, r2), (c3, v3, r3) = edges
    return f(xa, xb, zr, c0, v0, r0, c1, v1, r1, c2, v2, r2, c3, v3, r3)


def _elu(v):
    return jnp.where(v > 0, v, jnp.exp(v) - 1.0)


def _dst_block(x, s1, s2, w1, w2, wself, bias, wcat, wq, wk, wt):
    dot = lambda a, b: jnp.dot(a, b, preferred_element_type=jnp.float32)
    self_ft = dot(x, wself)
    nb1 = dot(s1, w1)
    nb2 = dot(s2, w2)
    q = dot(self_ft, wq)
    k1 = dot(nb1, wk)
    k2 = dot(nb2, wk)
    qs = dot(q, wt[_ATT:, :])
    e1 = _elu(dot(k1, wt[:_ATT, :]) + qs)
    e2 = _elu(dot(k2, wt[:_ATT, :]) + qs)
    m = jnp.maximum(e1, e2)
    x1 = jnp.exp(e1 - m)
    x2 = jnp.exp(e2 - m)
    inv = 1.0 / (x1 + x2)
    agg = nb1 * (x1 * inv) + nb2 * (x2 * inv)
    return dot(agg, wcat[:_D, :]) + dot(self_ft, wcat[_D:, :]) + bias


def _tc_post(xa, xb, s0, s1, s2, s3,
             Waa, Wab, wsa, ba, wca, wqa, wka, wta,
             Wba, Wbb, wsb, bb, wcb, wqb, wkb, wtb):
    B = 2000
    grid = (_N // B,)

    def row():
        return pl.BlockSpec((B, _D), lambda i: (i, 0))

    def full(a):
        nd = a.ndim
        return pl.BlockSpec(a.shape, lambda i, _nd=nd: (0,) * _nd)

    def tc_body(xa_r, xb_r, s0_r, s1_r, s2_r, s3_r,
                Waa_r, Wab_r, wsa_r, ba_r, wca_r, wqa_r, wka_r, wta_r,
                Wba_r, Wbb_r, wsb_r, bb_r, wcb_r, wqb_r, wkb_r, wtb_r,
                oa_r, ob_r):
        oa_r[...] = _dst_block(xa_r[...], s0_r[...], s1_r[...],
                               Waa_r[...], Wab_r[...], wsa_r[...], ba_r[...],
                               wca_r[...], wqa_r[...], wka_r[...], wta_r[...])
        ob_r[...] = _dst_block(xb_r[...], s2_r[...], s3_r[...],
                               Wba_r[...], Wbb_r[...], wsb_r[...], bb_r[...],
                               wcb_r[...], wqb_r[...], wkb_r[...], wtb_r[...])

    args = (xa, xb, s0, s1, s2, s3, Waa, Wab, wsa, ba, wca, wqa, wka, wta,
            Wba, Wbb, wsb, bb, wcb, wqb, wkb, wtb)
    in_specs = [row()] * 6 + [full(a) for a in args[6:]]
    out = pl.pallas_call(
        tc_body,
        grid=grid,
        in_specs=in_specs,
        out_specs=[row(), row()],
        out_shape=[jax.ShapeDtypeStruct((_N, _D), jnp.float32)] * 2,
    )(*args)
    return out[0], out[1]


def kernel(x_a, x_b, edge_index_aa, adj_val_aa, edge_index_ab, adj_val_ab,
           edge_index_ba, adj_val_ba, edge_index_bb, adj_val_bb,
           Wrel_a_a, Wrel_a_b, wself_a, bias_a, wcat_a, wquery_a, wkeys_a,
           watt_a, Wrel_b_a, Wrel_b_b, wself_b, bias_b, wcat_b, wquery_b,
           wkeys_b, watt_b):
    def prep(ei, v):
        return (ei[1], v, ei[0])

    edges = [prep(edge_index_aa, adj_val_aa), prep(edge_index_ab, adj_val_ab),
             prep(edge_index_ba, adj_val_ba), prep(edge_index_bb, adj_val_bb)]
    s0, s1, s2, s3 = _sc_spmm(x_a, x_b, edges)
    return _tc_post(x_a, x_b, s0, s1, s2, s3,
                    Wrel_a_a, Wrel_a_b, wself_a, bias_a, wcat_a, wquery_a,
                    wkeys_a, watt_a, Wrel_b_a, Wrel_b_b, wself_b, bias_b,
                    wcat_b, wquery_b, wkeys_b, watt_b)


# speedup vs baseline: 2.3136x; 2.3136x over previous
import jax
import jax.numpy as jnp
from jax import lax
from jax.experimental import pallas as pl
from jax.experimental.pallas import tpu as pltpu
from jax.experimental.pallas import tpu_sc as plsc

_N = 10000
_D = 128
_ATT = 64
_E = 320000
_L = 16
_NS = 16
_CPT = 156
_TAIL = _E - _CPT * _NS * 128
_ZR = 624


def _sc_spmm(xa, xb, edges):
    mesh = plsc.VectorSubcoreMesh(core_axis_name="c", subcore_axis_name="s")
    zr = jnp.zeros((_ZR, _D), jnp.float32)
    out_type = tuple(jax.ShapeDtypeStruct((_N, _D), jnp.float32) for _ in range(4))
    scratch = [
        pltpu.VMEM_SHARED((_N, _D), jnp.float32),
        pltpu.VMEM((128, _D), jnp.float32),
        pltpu.VMEM((2, 128), jnp.int32),
        pltpu.VMEM((2, 128), jnp.float32),
        pltpu.VMEM((2, 128), jnp.int32),
        pltpu.SemaphoreType.DMA,
        pltpu.SemaphoreType.DMA,
    ]

    def body(xa_h, xb_h, z_h,
             c0, v0, r0, c1, v1, r1, c2, v2, r2, c3, v3, r3,
             o0, o1, o2, o3,
             acc, gbuf, colv, valv, rowv, sem, semi):
        cid = lax.axis_index("c")
        sid = lax.axis_index("s")

        def do_rel(x_h, cols_h, vals_h, rows_h, out_h):
            pltpu.sync_copy(z_h, acc.at[pl.ds(sid * _ZR, _ZR)])

            @pl.when(sid == 0)
            def _():
                pltpu.sync_copy(z_h.at[pl.ds(0, 16)],
                                acc.at[pl.ds(_NS * _ZR, 16)])

            plsc.subcore_barrier()

            def idx_load(base, slot, copy):
                copy(cols_h.at[pl.ds(base, 128)], colv.at[slot])
                copy(vals_h.at[pl.ds(base, 128)], valv.at[slot])
                copy(rows_h.at[pl.ds(base, 128)], rowv.at[slot])

            def body_chunk(b, do_pf, prefetch_base):
                pltpu.async_copy(x_h.at[colv.at[b]], gbuf, sem).wait()

                @pl.when(do_pf)
                def _():
                    idx_load(pl.multiple_of(prefetch_base, 128), 1 - b,
                             lambda s_, d_: pltpu.async_copy(s_, d_, semi))

                def grp(i, c2_):
                    off = pl.multiple_of(i * _L, _L)
                    v16 = valv[b, pl.ds(off, _L)]
                    for e in range(_L):
                        row = off + e
                        sv = v16[e]
                        for f in range(_D // _L):
                            sl = pl.ds(f * _L, _L)
                            gbuf[row, sl] = gbuf[row, sl] * sv
                    return c2_

                lax.fori_loop(0, 128 // _L, grp, 0)
                pltpu.sync_copy(gbuf, acc.at[rowv.at[b]], add=True)

                @pl.when(do_pf)
                def _():
                    idx_load(0, 1 - b,
                             lambda s_, d_:
                             pltpu.make_async_copy(s_, d_, semi).wait())

            j0 = sid * _CPT * 128
            idx_load(j0, 0, pltpu.sync_copy)

            def step(k, carry):
                nxt = jnp.where(k + 1 < _CPT, j0 + (k + 1) * 128, j0)
                body_chunk(k & 1, k + 1 < _CPT, nxt)
                return carry

            lax.fori_loop(0, _CPT, step, 0)

            @pl.when(sid < _TAIL // 128)
            def _():
                base = _CPT * _NS * 128 + sid * 128
                idx_load(base, 0, pltpu.sync_copy)
                body_chunk(0, sid < 0, jnp.int32(0))

            plsc.subcore_barrier()
            pltpu.sync_copy(acc.at[pl.ds(sid * _ZR, _ZR)],
                            out_h.at[pl.ds(sid * _ZR, _ZR)])

            @pl.when(sid == 0)
            def _():
                pltpu.sync_copy(acc.at[pl.ds(_NS * _ZR, 16)],
                                out_h.at[pl.ds(_NS * _ZR, 16)])

            plsc.subcore_barrier()

        @pl.when(cid == 0)
        def _():
            do_rel(xa_h, c0, v0, r0, o0)
            do_rel(xb_h, c1, v1, r1, o1)

        @pl.when(cid == 1)
        def _():
            do_rel(xa_h, c2, v2, r2, o2)
            do_rel(xb_h, c3, v3, r3, o3)

    f = pl.kernel(body, out_type=out_type, mesh=mesh, scratch_types=scratch)
    (c0, v0, r0), (c1, v1, r1), (c2, v2, r2), (c3, v3, r3) = edges
    return f(xa, xb, zr, c0, v0, r0, c1, v1, r1, c2, v2, r2, c3, v3, r3)


def _elu(v):
    return jnp.where(v > 0, v, jnp.exp(v) - 1.0)


def _dst_block(x, s1, s2, w1, w2, wself, bias, wcat, wq, wk, wt):
    dot = lambda a, b: jnp.dot(a, b, preferred_element_type=jnp.float32)
    self_ft = dot(x, wself)
    nb1 = dot(s1, w1)
    nb2 = dot(s2, w2)
    q = dot(self_ft, wq)
    k1 = dot(nb1, wk)
    k2 = dot(nb2, wk)
    qs = dot(q, wt[_ATT:, :])
    e1 = _elu(dot(k1, wt[:_ATT, :]) + qs)
    e2 = _elu(dot(k2, wt[:_ATT, :]) + qs)
    m = jnp.maximum(e1, e2)
    x1 = jnp.exp(e1 - m)
    x2 = jnp.exp(e2 - m)
    inv = 1.0 / (x1 + x2)
    agg = nb1 * (x1 * inv) + nb2 * (x2 * inv)
    return dot(agg, wcat[:_D, :]) + dot(self_ft, wcat[_D:, :]) + bias


def _tc_post(xa, xb, s0, s1, s2, s3,
             Waa, Wab, wsa, ba, wca, wqa, wka, wta,
             Wba, Wbb, wsb, bb, wcb, wqb, wkb, wtb):
    B = 2000
    grid = (_N // B,)

    def row():
        return pl.BlockSpec((B, _D), lambda i: (i, 0))

    def full(a):
        nd = a.ndim
        return pl.BlockSpec(a.shape, lambda i, _nd=nd: (0,) * _nd)

    def tc_body(xa_r, xb_r, s0_r, s1_r, s2_r, s3_r,
                Waa_r, Wab_r, wsa_r, ba_r, wca_r, wqa_r, wka_r, wta_r,
                Wba_r, Wbb_r, wsb_r, bb_r, wcb_r, wqb_r, wkb_r, wtb_r,
                oa_r, ob_r):
        oa_r[...] = _dst_block(xa_r[...], s0_r[...], s1_r[...],
                               Waa_r[...], Wab_r[...], wsa_r[...], ba_r[...],
                               wca_r[...], wqa_r[...], wka_r[...], wta_r[...])
        ob_r[...] = _dst_block(xb_r[...], s2_r[...], s3_r[...],
                               Wba_r[...], Wbb_r[...], wsb_r[...], bb_r[...],
                               wcb_r[...], wqb_r[...], wkb_r[...], wtb_r[...])

    args = (xa, xb, s0, s1, s2, s3, Waa, Wab, wsa, ba, wca, wqa, wka, wta,
            Wba, Wbb, wsb, bb, wcb, wqb, wkb, wtb)
    in_specs = [row()] * 6 + [full(a) for a in args[6:]]
    out = pl.pallas_call(
        tc_body,
        grid=grid,
        in_specs=in_specs,
        out_specs=[row(), row()],
        out_shape=[jax.ShapeDtypeStruct((_N, _D), jnp.float32)] * 2,
    )(*args)
    return out[0], out[1]


def kernel(x_a, x_b, edge_index_aa, adj_val_aa, edge_index_ab, adj_val_ab,
           edge_index_ba, adj_val_ba, edge_index_bb, adj_val_bb,
           Wrel_a_a, Wrel_a_b, wself_a, bias_a, wcat_a, wquery_a, wkeys_a,
           watt_a, Wrel_b_a, Wrel_b_b, wself_b, bias_b, wcat_b, wquery_b,
           wkeys_b, watt_b):
    def prep(ei, v):
        return (ei[1], v, ei[0])

    edges = [prep(edge_index_aa, adj_val_aa), prep(edge_index_ab, adj_val_ab),
             prep(edge_index_ba, adj_val_ba), prep(edge_index_bb, adj_val_bb)]
    s0, s1, s2, s3 = _sc_spmm(x_a, x_b, edges)
    return _tc_post(x_a, x_b, s0, s1, s2, s3,
                    Wrel_a_a, Wrel_a_b, wself_a, bias_a, wcat_a, wquery_a,
                    wkeys_a, watt_a, Wrel_b_a, Wrel_b_b, wself_b, bias_b,
                    wcat_b, wquery_b, wkeys_b, watt_b)
